# baseline (speedup 1.0000x reference)
import jax
import jax.numpy as jnp
from jax.experimental import pallas as pl
from jax.experimental.pallas import tpu as pltpu

_TB = 16384


def _make_body(s_dim, a_dim):
    row_bh = s_dim + a_dim
    row_wo = row_bh + 1
    row_bo = row_wo + 1

    def body(s_ref, a_ref, w_ref, q_ref):
        x = jnp.concatenate(
            [s_ref[...].astype(jnp.bfloat16), a_ref[...].astype(jnp.bfloat16)],
            axis=1)
        w = w_ref[:row_bh, :].astype(jnp.bfloat16)
        h = jnp.dot(x, w, preferred_element_type=jnp.float32)
        h = jnp.maximum(h + w_ref[row_bh:row_bh + 1, :], 0.0)
        q = jnp.sum(h * w_ref[row_wo:row_wo + 1, :], axis=-1, keepdims=True)
        q_ref[...] = q + w_ref[row_bo:row_bo + 1, 0:1]

    return body


def kernel(s, a, w_all):
    B, s_dim = s.shape
    a_dim = a.shape[1]
    k_rows, hidden = w_all.shape

    tb = min(_TB, B) if B % min(_TB, B) == 0 else _TB
    pad = (-B) % tb
    if pad:
        s = jnp.pad(s, ((0, pad), (0, 0)))
        a = jnp.pad(a, ((0, pad), (0, 0)))
    bp = B + pad

    q = pl.pallas_call(
        _make_body(s_dim, a_dim),
        out_shape=jax.ShapeDtypeStruct((bp, 1), jnp.float32),
        grid=(bp // tb,),
        in_specs=[
            pl.BlockSpec((tb, s_dim), lambda i: (i, 0)),
            pl.BlockSpec((tb, a_dim), lambda i: (i, 0)),
            pl.BlockSpec((k_rows, hidden), lambda i: (0, 0)),
        ],
        out_specs=pl.BlockSpec((tb, 1), lambda i: (i, 0)),
        compiler_params=pltpu.CompilerParams(
            dimension_semantics=("parallel",),
            vmem_limit_bytes=64 << 20,
        ),
    )(s, a, w_all)
    return q[:B]


# speedup vs baseline: 1.0169x; 1.0169x over previous
import jax
import jax.numpy as jnp
from jax.experimental import pallas as pl
from jax.experimental.pallas import tpu as pltpu

_TB = 8192
_HB = _TB // 2


def _make_body(s_dim, a_dim):
    row_bh = s_dim + a_dim
    row_wo = row_bh + 1
    row_bo = row_wo + 1

    def half(s_ref, a_ref, w_ref, wq, bq):
        x = jnp.concatenate(
            [s_ref[...].astype(jnp.bfloat16), a_ref[...].astype(jnp.bfloat16)],
            axis=1)
        h = jnp.dot(x, w_ref[:row_bh, :].astype(jnp.bfloat16),
                    preferred_element_type=jnp.float32)
        h = jnp.maximum(h + w_ref[row_bh:row_bh + 1, :], 0.0)
        return jnp.sum(h * wq, axis=-1, keepdims=True) + bq

    def body(s0_ref, s1_ref, a0_ref, a1_ref, w_ref, q_ref):
        wq = w_ref[row_wo:row_wo + 1, :]
        bq = w_ref[row_bo:row_bo + 1, 0:1]
        q_ref[:_HB, :] = half(s0_ref, a0_ref, w_ref, wq, bq)
        q_ref[_HB:, :] = half(s1_ref, a1_ref, w_ref, wq, bq)

    return body


def kernel(s, a, w_all):
    B, s_dim = s.shape
    a_dim = a.shape[1]
    k_rows, hidden = w_all.shape

    tb = _TB if B % _TB == 0 else min(B, 512)
    hb = tb // 2
    pad = (-B) % tb
    if pad:
        s = jnp.pad(s, ((0, pad), (0, 0)))
        a = jnp.pad(a, ((0, pad), (0, 0)))
    bp = B + pad

    if tb != _TB:
        def small_body(s_ref, a_ref, w_ref, q_ref):
            row_bh = s_dim + a_dim
            x = jnp.concatenate(
                [s_ref[...].astype(jnp.bfloat16),
                 a_ref[...].astype(jnp.bfloat16)], axis=1)
            h = jnp.dot(x, w_ref[:row_bh, :].astype(jnp.bfloat16),
                        preferred_element_type=jnp.float32)
            h = jnp.maximum(h + w_ref[row_bh:row_bh + 1, :], 0.0)
            q = jnp.sum(h * w_ref[row_bh + 1:row_bh + 2, :], axis=-1,
                        keepdims=True)
            q_ref[...] = q + w_ref[row_bh + 2:row_bh + 3, 0:1]

        q = pl.pallas_call(
            small_body,
            out_shape=jax.ShapeDtypeStruct((bp, 1), jnp.float32),
            grid=(bp // tb,),
            in_specs=[
                pl.BlockSpec((tb, s_dim), lambda i: (i, 0)),
                pl.BlockSpec((tb, a_dim), lambda i: (i, 0)),
                pl.BlockSpec((k_rows, hidden), lambda i: (0, 0)),
            ],
            out_specs=pl.BlockSpec((tb, 1), lambda i: (i, 0)),
            compiler_params=pltpu.CompilerParams(
                dimension_semantics=("parallel",),
                vmem_limit_bytes=64 << 20,
            ),
        )(s, a, w_all)
        return q[:B]

    q = pl.pallas_call(
        _make_body(s_dim, a_dim),
        out_shape=jax.ShapeDtypeStruct((bp, 1), jnp.float32),
        grid=(bp // tb,),
        in_specs=[
            pl.BlockSpec((hb, s_dim), lambda i: (2 * i, 0)),
            pl.BlockSpec((hb, s_dim), lambda i: (2 * i + 1, 0)),
            pl.BlockSpec((hb, a_dim), lambda i: (2 * i, 0)),
            pl.BlockSpec((hb, a_dim), lambda i: (2 * i + 1, 0)),
            pl.BlockSpec((k_rows, hidden), lambda i: (0, 0)),
        ],
        out_specs=pl.BlockSpec((tb, 1), lambda i: (i, 0)),
        compiler_params=pltpu.CompilerParams(
            dimension_semantics=("parallel",),
            vmem_limit_bytes=64 << 20,
        ),
    )(s, s, a, a, w_all)
    return q[:B]


# speedup vs baseline: 1.8625x; 1.8316x over previous
import jax
import jax.numpy as jnp
from jax import lax
from jax.experimental import pallas as pl
from jax.experimental.pallas import tpu as pltpu

_TB = 8192
_HB = _TB // 2


def _body(s0_ref, s1_ref, a0_ref, a1_ref, wt_ref, bcol_ref, wocol_ref,
          bo_ref, q_ref):
    bo = bo_ref[0, 0]
    for j, (sr, ar) in enumerate(((s0_ref, a0_ref), (s1_ref, a1_ref))):
        x = jnp.concatenate(
            [sr[...].astype(jnp.bfloat16), ar[...].astype(jnp.bfloat16)],
            axis=1)
        ht = lax.dot_general(wt_ref[...], x, (((1,), (1,)), ((), ())),
                             preferred_element_type=jnp.float32)
        ht = jnp.maximum(ht + bcol_ref[...], 0.0)
        q = jnp.sum(ht * wocol_ref[...], axis=0, keepdims=True)
        q_ref[0:1, j * _HB:(j + 1) * _HB] = q + bo


def kernel(s, a, w_all):
    B, s_dim = s.shape
    a_dim = a.shape[1]
    k_rows, hidden = w_all.shape
    row_bh = s_dim + a_dim
    row_wo = row_bh + 1
    row_bo = row_wo + 1

    wt = jnp.transpose(w_all[:row_bh, :]).astype(jnp.bfloat16)
    bcol = jnp.transpose(w_all[row_bh:row_bh + 1, :])
    wocol = jnp.transpose(w_all[row_wo:row_wo + 1, :])
    bo = w_all[row_bo:row_bo + 1, 0:1]

    tb = _TB if B % _TB == 0 else min(B, 512)
    pad = (-B) % tb
    if pad:
        s = jnp.pad(s, ((0, pad), (0, 0)))
        a = jnp.pad(a, ((0, pad), (0, 0)))
    bp = B + pad

    if tb != _TB:
        def small_body(s_ref, a_ref, wt_ref, bcol_ref, wocol_ref, bo_ref,
                       q_ref):
            x = jnp.concatenate(
                [s_ref[...].astype(jnp.bfloat16),
                 a_ref[...].astype(jnp.bfloat16)], axis=1)
            ht = lax.dot_general(wt_ref[...], x, (((1,), (1,)), ((), ())),
                                 preferred_element_type=jnp.float32)
            ht = jnp.maximum(ht + bcol_ref[...], 0.0)
            q = jnp.sum(ht * wocol_ref[...], axis=0, keepdims=True)
            q_ref[...] = q + bo_ref[0, 0]

        q = pl.pallas_call(
            small_body,
            out_shape=jax.ShapeDtypeStruct((1, bp), jnp.float32),
            grid=(bp // tb,),
            in_specs=[
                pl.BlockSpec((tb, s_dim), lambda i: (i, 0)),
                pl.BlockSpec((tb, a_dim), lambda i: (i, 0)),
                pl.BlockSpec((hidden, row_bh), lambda i: (0, 0)),
                pl.BlockSpec((hidden, 1), lambda i: (0, 0)),
                pl.BlockSpec((hidden, 1), lambda i: (0, 0)),
                pl.BlockSpec((1, 1), lambda i: (0, 0)),
            ],
            out_specs=pl.BlockSpec((1, tb), lambda i: (0, i)),
            compiler_params=pltpu.CompilerParams(
                dimension_semantics=("arbitrary",),
                vmem_limit_bytes=64 << 20,
            ),
        )(s, a, wt, bcol, wocol, bo)
        return jnp.reshape(q[:, :B], (B, 1))

    hb = tb // 2
    q = pl.pallas_call(
        _body,
        out_shape=jax.ShapeDtypeStruct((1, bp), jnp.float32),
        grid=(bp // tb,),
        in_specs=[
            pl.BlockSpec((hb, s_dim), lambda i: (2 * i, 0)),
            pl.BlockSpec((hb, s_dim), lambda i: (2 * i + 1, 0)),
            pl.BlockSpec((hb, a_dim), lambda i: (2 * i, 0)),
            pl.BlockSpec((hb, a_dim), lambda i: (2 * i + 1, 0)),
            pl.BlockSpec((hidden, row_bh), lambda i: (0, 0)),
            pl.BlockSpec((hidden, 1), lambda i: (0, 0)),
            pl.BlockSpec((hidden, 1), lambda i: (0, 0)),
            pl.BlockSpec((1, 1), lambda i: (0, 0)),
        ],
        out_specs=pl.BlockSpec((1, tb), lambda i: (0, i)),
        compiler_params=pltpu.CompilerParams(
            dimension_semantics=("arbitrary",),
            vmem_limit_bytes=64 << 20,
        ),
    )(s, s, a, a, wt, bcol, wocol, bo)
    return jnp.reshape(q, (B, 1)) if not pad else jnp.reshape(q[:, :B], (B, 1))


# speedup vs baseline: 2.2878x; 1.2283x over previous
import jax
import jax.numpy as jnp
from jax import lax
from jax.experimental import pallas as pl
from jax.experimental.pallas import tpu as pltpu

_TB = 8192


def _make_body(s_dim, a_dim):
    row_bh = s_dim + a_dim
    row_wo = row_bh + 1
    row_bo = row_wo + 1

    def body(s_ref, a_ref, wt_ref, q_ref):
        x = jnp.concatenate(
            [s_ref[...].astype(jnp.bfloat16), a_ref[...].astype(jnp.bfloat16)],
            axis=1)
        wt = wt_ref[:, :row_bh].astype(jnp.bfloat16)
        ht = lax.dot_general(wt, x, (((1,), (1,)), ((), ())),
                             preferred_element_type=jnp.float32)
        ht = jnp.maximum(ht + wt_ref[:, row_bh:row_bh + 1], 0.0)
        q = jnp.sum(ht * wt_ref[:, row_wo:row_wo + 1], axis=0, keepdims=True)
        q_ref[...] = q + wt_ref[0, row_bo]

    return body


def kernel(s, a, w_all):
    B, s_dim = s.shape
    a_dim = a.shape[1]
    k_rows, hidden = w_all.shape

    wt_all = jnp.transpose(w_all)

    tb = _TB if B % _TB == 0 else min(B, 512)
    pad = (-B) % tb
    if pad:
        s = jnp.pad(s, ((0, pad), (0, 0)))
        a = jnp.pad(a, ((0, pad), (0, 0)))
    bp = B + pad

    q = pl.pallas_call(
        _make_body(s_dim, a_dim),
        out_shape=jax.ShapeDtypeStruct((1, bp), jnp.float32),
        grid=(bp // tb,),
        in_specs=[
            pl.BlockSpec((tb, s_dim), lambda i: (i, 0)),
            pl.BlockSpec((tb, a_dim), lambda i: (i, 0)),
            pl.BlockSpec((hidden, k_rows), lambda i: (0, 0)),
        ],
        out_specs=pl.BlockSpec((1, tb), lambda i: (0, i)),
        compiler_params=pltpu.CompilerParams(
            dimension_semantics=("arbitrary",),
            vmem_limit_bytes=64 << 20,
        ),
    )(s, a, wt_all)
    return jnp.reshape(q[:, :B], (B, 1))
